# baseline (speedup 1.0000x reference)
import functools

import jax
import jax.numpy as jnp
from jax import lax
from jax.experimental import pallas as pl
from jax.experimental.pallas import tpu as pltpu
from jax.experimental.pallas import tpu_sc as plsc

BATCH = 16384
HIST = 50
DIM = 16
NUM_ACT = 100000
NUM_WORKERS = 32
COLS_W = BATCH // NUM_WORKERS
CB = 256
NSUB = COLS_W // CB
NCHUNKS = HIST * NSUB
NBUF = 2
STAGE = NUM_ACT // 16

_mesh = plsc.VectorSubcoreMesh(core_axis_name="c", subcore_axis_name="s")


@functools.partial(
    pl.kernel,
    mesh=_mesh,
    out_type=jax.ShapeDtypeStruct((HIST * DIM, BATCH), jnp.float32),
    scratch_types=[
        pltpu.VMEM_SHARED((NUM_ACT, DIM), jnp.float32),
        pltpu.VMEM((NBUF, CB), jnp.int32),
        pltpu.VMEM((NBUF, CB, DIM), jnp.float32),
        pltpu.VMEM((NBUF, DIM, CB), jnp.float32),
        pltpu.SemaphoreType.DMA,
        pltpu.SemaphoreType.DMA,
        pltpu.SemaphoreType.DMA,
        pltpu.SemaphoreType.DMA,
    ],
    compiler_params=pltpu.CompilerParams(use_tc_tiling_on_sc=False,
                                         needs_layout_passes=False),
)
def _gather_cm(idx_hbm, table_hbm, out_hbm, table_sp, idx_v, rows_v,
               col_v, g0, g1, s0, s1):
    cid = lax.axis_index("c")
    sid = lax.axis_index("s")
    wid = sid * 2 + cid
    b0 = wid * COLS_W
    gsem = (g0, g1)
    ssem = (s0, s1)

    pltpu.sync_copy(table_hbm.at[pl.ds(sid * STAGE, STAGE)],
                    table_sp.at[pl.ds(sid * STAGE, STAGE)])
    plsc.subcore_barrier()

    lanes = lax.iota(jnp.int32, DIM)

    def idx_load(t, b):
        e, half = t // NSUB, t % NSUB
        pltpu.sync_copy(idx_hbm.at[e, pl.ds(b0 + half * CB, CB)], idx_v.at[b])

    def gather_start(t, b):
        pltpu.async_copy(table_sp.at[idx_v.at[b]], rows_v.at[b], gsem[b])

    def gather_drain(b):
        pltpu.make_async_copy(
            out_hbm.at[pl.ds(0, CB), pl.ds(0, DIM)], rows_v.at[b],
            gsem[b]).wait()

    def transpose(b):
        def block(a0, carry):
            rowids = a0 + lanes
            for f in range(DIM):
                col_v[b, f, pl.ds(a0, DIM)] = plsc.load_gather(
                    rows_v.at[b], [rowids, jnp.full((DIM,), f, jnp.int32)])
            return carry

        lax.fori_loop(0, CB // DIM, lambda i, c: block(i * DIM, c), 0)

    def store_start(t, b):
        e, half = t // NSUB, t % NSUB
        pltpu.async_copy(
            col_v.at[b],
            out_hbm.at[pl.ds(e * DIM, DIM), pl.ds(b0 + half * CB, CB)],
            ssem[b])

    def store_drain(b):
        pltpu.make_async_copy(
            col_v.at[b], out_hbm.at[pl.ds(0, DIM), pl.ds(0, CB)],
            ssem[b]).wait()

    for b in range(NBUF):
        idx_load(b, b)
        gather_start(b, b)

    def chunk_mid(t, b):
        gather_drain(b)
        transpose(b)
        store_start(t, b)
        idx_load(t + NBUF, b)
        gather_start(t + NBUF, b)

    for b in range(NBUF):
        chunk_mid(b, b)

    def body(i, carry):
        t = i * NBUF
        for b in range(NBUF):
            store_drain(b)
            chunk_mid(t + b, b)
        return carry

    lax.fori_loop(1, NCHUNKS // NBUF - 1, body, 0)

    for b in range(NBUF):
        t = NCHUNKS - NBUF + b
        store_drain(b)
        gather_drain(b)
        transpose(b)
        store_start(t, b)
    for b in range(NBUF):
        store_drain(b)


def kernel(action_history, embedding_weight):
    idx_cm = action_history.T.astype(jnp.int32)
    out_cm = _gather_cm(idx_cm, embedding_weight)
    return out_cm.T


# speedup vs baseline: 1.6739x; 1.6739x over previous
import functools

import jax
import jax.numpy as jnp
from jax import lax
from jax.experimental import pallas as pl
from jax.experimental.pallas import tpu as pltpu
from jax.experimental.pallas import tpu_sc as plsc

BATCH = 16384
HIST = 50
DIM = 16
TOTAL = BATCH * HIST
NUM_WORKERS = 32
PER_WORKER = TOTAL // NUM_WORKERS
CHUNK = 2560
NCHUNKS = PER_WORKER // CHUNK
NBUF = 2

_mesh = plsc.VectorSubcoreMesh(core_axis_name="c", subcore_axis_name="s")


@functools.partial(
    pl.kernel,
    mesh=_mesh,
    out_type=jax.ShapeDtypeStruct((TOTAL, DIM), jnp.float32),
    scratch_types=[
        pltpu.VMEM((PER_WORKER,), jnp.int32),
        pltpu.VMEM((NBUF, CHUNK, DIM), jnp.float32),
        pltpu.SemaphoreType.DMA,
        pltpu.SemaphoreType.DMA,
        pltpu.SemaphoreType.DMA,
        pltpu.SemaphoreType.DMA,
    ],
    compiler_params=pltpu.CompilerParams(use_tc_tiling_on_sc=False),
)
def _gather_rows(idx_hbm, table_hbm, out_hbm, idx_v, rows_v, g0, g1, s0, s1):
    wid = lax.axis_index("s") * 2 + lax.axis_index("c")
    base = wid * PER_WORKER
    gsem = (g0, g1)
    ssem = (s0, s1)

    pltpu.sync_copy(idx_hbm.at[pl.ds(base, PER_WORKER)], idx_v)

    def gather_start(g):
        b = g % NBUF
        return pltpu.async_copy(
            table_hbm.at[idx_v.at[pl.ds(g * CHUNK, CHUNK)]],
            rows_v.at[b], gsem[b])

    def store_start(g):
        b = g % NBUF
        return pltpu.async_copy(
            rows_v.at[b], out_hbm.at[pl.ds(base + g * CHUNK, CHUNK)], ssem[b])

    gh = {0: gather_start(0)}
    sh = {}
    for g in range(NCHUNKS):
        if g + 1 < NCHUNKS:
            if g >= 1:
                sh[g - 1].wait()
            gh[g + 1] = gather_start(g + 1)
        gh[g].wait()
        sh[g] = store_start(g)
    sh[NCHUNKS - 2].wait()
    sh[NCHUNKS - 1].wait()


def kernel(action_history, embedding_weight):
    idx = action_history.reshape(-1).astype(jnp.int32)
    out = _gather_rows(idx, embedding_weight)
    return out.reshape(action_history.shape[0], HIST * DIM)
